# baseline (speedup 1.0000x reference)
import jax
import jax.numpy as jnp
from jax import lax
from jax.experimental import pallas as pl
from jax.experimental.pallas import tpu as pltpu
from jax.experimental.pallas import tpu_sc as plsc

NUM_USERS = 12500
NUM_ITEMS = 37500
N = NUM_USERS + NUM_ITEMS
E = 800000
D = 64
DH = D // 2
NC = 2
NS = 16
LANES = 16

SUB = 128
NJ = 2
CH = SUB * NJ
NSLOT = 3
NU = 66
NCHUNK = NSLOT * NU
EPW = CH * NCHUNK
EPAD = NS * EPW

RCH = N // NS

_mesh = plsc.VectorSubcoreMesh(
    core_axis_name="c", subcore_axis_name="s", num_cores=NC, num_subcores=NS
)


def _gcn_body(x0_hbm, ep_hbm, x1_hbm, x2_hbm, x3_hbm,
              epv, rowsv, acc, sem_g, sem_s, sem_i):
    c = lax.axis_index("c")
    s = lax.axis_index("s")
    cbase = s * NCHUNK

    def run_layer(x_hbm, out_hbm):
        xc = x_hbm.at[c]
        outc = out_hbm.at[c]
        _layer(xc, outc, ep_hbm, epv, rowsv, acc, sem_g, sem_s, sem_i,
               c, s, cbase)

    run_layer(x0_hbm, x1_hbm)
    run_layer(x1_hbm, x2_hbm)
    run_layer(x2_hbm, x3_hbm)


def _layer(xc, outc, ep_hbm, epv, rowsv, acc, sem_g, sem_s, sem_i,
           c, s, cbase):
    @pl.loop(0, CH)
    def _zero_rows(r):
        for k in range(DH // LANES):
            rowsv[0, r, pl.ds(k * LANES, LANES)] = jnp.zeros(
                (LANES,), jnp.float32)

    zstart = s * RCH
    n_full, rem = RCH // CH, RCH % CH
    for z in range(n_full):
        pltpu.sync_copy(rowsv.at[0], acc.at[pl.ds(zstart + z * CH, CH)])
    if rem:
        pltpu.sync_copy(rowsv.at[0, pl.ds(0, rem)],
                        acc.at[pl.ds(zstart + n_full * CH, rem)])
    plsc.subcore_barrier()

    def fire_idx(i, p):
        pltpu.async_copy(ep_hbm.at[cbase + i], epv.at[p], sem_i[p])

    def wait_idx(p):
        pltpu.make_async_copy(ep_hbm.at[0], epv.at[p], sem_i[p]).wait()

    def fire_gather(p):
        for j in range(NJ):
            pltpu.async_copy(xc.at[epv.at[p, 0, j]],
                             rowsv.at[p, pl.ds(j * SUB, SUB)], sem_g[p])

    def wait_gather(p):
        for j in range(NJ):
            pltpu.make_async_copy(xc.at[epv.at[p, 0, j]],
                                  rowsv.at[p, pl.ds(j * SUB, SUB)],
                                  sem_g[p]).wait()

    def fire_scatter(p):
        for j in range(NJ):
            pltpu.async_copy(rowsv.at[p, pl.ds(j * SUB, SUB)],
                             acc.at[epv.at[p, 1, j]], sem_s[p], add=True)

    def wait_scatter(p):
        for j in range(NJ):
            pltpu.make_async_copy(rowsv.at[p, pl.ds(j * SUB, SUB)],
                                  acc.at[epv.at[p, 1, j]], sem_s[p]).wait()

    def compute(p):
        for j in range(NJ):
            @plsc.parallel_loop(0, SUB // LANES)
            def _group(g, j=j):
                w16 = plsc.bitcast(epv[p, 2, j, pl.ds(g * LANES, LANES)],
                                   jnp.float32)
                e0 = j * SUB + g * LANES
                for l in range(LANES):
                    wb = lax.broadcast(w16[l], (LANES,))
                    for k in range(DH // LANES):
                        sl = pl.ds(k * LANES, LANES)
                        rowsv[p, e0 + l, sl] = rowsv[p, e0 + l, sl] * wb

    def step(i, d, u):
        p = d
        q = (d + 1) % NSLOT
        wait_gather(p)
        if d < NSLOT - 1:
            @pl.when(u > 0)
            def _drain():
                wait_scatter(q)
        else:
            wait_scatter(q)
        if d < NSLOT - 1:
            wait_idx(q)
            fire_gather(q)
        else:
            @pl.when(u < NU - 1)
            def _next_gather():
                wait_idx(q)
                fire_gather(q)
        compute(p)
        fire_scatter(p)
        @pl.when(u < NU - 1)
        def _prefetch():
            fire_idx(i + NSLOT, p)

    for p in range(NSLOT):
        fire_idx(p, p)
    wait_idx(0)
    fire_gather(0)

    @pl.loop(0, NU)
    def _ring(u):
        for d in range(NSLOT):
            step(u * NSLOT + d, d, u)

    for r in range(1, NSLOT):
        wait_scatter((NCHUNK - NSLOT + r) % NSLOT)
    plsc.subcore_barrier()

    pltpu.sync_copy(acc.at[pl.ds(s * RCH, RCH)],
                    outc.at[pl.ds(s * RCH, RCH)])
    plsc.subcore_barrier()


def _propagate3(x0h, epack):
    kern = pl.kernel(
        _gcn_body,
        out_type=[jax.ShapeDtypeStruct((NC, N, DH), jnp.float32)] * 3,
        mesh=_mesh,
        scratch_types=[
            pltpu.VMEM((NSLOT, 3, NJ, SUB), jnp.int32),
            pltpu.VMEM((NSLOT, CH, DH), jnp.float32),
            pltpu.VMEM_SHARED((N, DH), jnp.float32),
            [pltpu.SemaphoreType.DMA] * NSLOT,
            [pltpu.SemaphoreType.DMA] * NSLOT,
            [pltpu.SemaphoreType.DMA] * NSLOT,
        ],
        compiler_params=pltpu.CompilerParams(
            use_tc_tiling_on_sc=False, needs_layout_passes=False),
    )
    return kern(x0h, epack)


def _mean_body(a_ref, b_ref, c_ref, d_ref, o_ref):
    o_ref[...] = (a_ref[...] + b_ref[...] + c_ref[...] + d_ref[...]) * 0.25


def _mean4(x0h, x1h, x2h, x3h):
    blk = 2000
    spec = pl.BlockSpec((1, blk, DH), lambda h, i: (h, i, 0))
    return pl.pallas_call(
        _mean_body,
        grid=(NC, N // blk),
        in_specs=[spec] * 4,
        out_specs=spec,
        out_shape=jax.ShapeDtypeStruct((NC, N, DH), jnp.float32),
    )(x0h, x1h, x2h, x3h)


def kernel(user_emb, item_emb, edge_index, edge_weight):
    x0 = jnp.concatenate([user_emb, item_emb], axis=0)
    x0h = jnp.stack([x0[:, :DH], x0[:, DH:]], axis=0)
    src = edge_index[0]
    dst = edge_index[1]

    pad = EPAD - E
    ar = jnp.arange(pad, dtype=jnp.int32)
    src_p = jnp.concatenate([src, (ar * 61) % N]).reshape(-1, NJ, SUB)
    dst_p = jnp.concatenate([dst, (ar * 97) % N]).reshape(-1, NJ, SUB)
    w_p = lax.bitcast_convert_type(
        jnp.concatenate([edge_weight, jnp.zeros((pad,), jnp.float32)]),
        jnp.int32).reshape(-1, NJ, SUB)
    epack = jnp.stack([src_p, dst_p, w_p], axis=1)

    x1h, x2h, x3h = _propagate3(x0h, epack)
    fh = _mean4(x0h, x1h, x2h, x3h)
    final = jnp.concatenate([fh[0], fh[1]], axis=1)
    return final[:NUM_USERS], final[NUM_USERS:]


# speedup vs baseline: 1.0310x; 1.0310x over previous
import jax
import jax.numpy as jnp
from jax import lax
from jax.experimental import pallas as pl
from jax.experimental.pallas import tpu as pltpu
from jax.experimental.pallas import tpu_sc as plsc

NUM_USERS = 12500
NUM_ITEMS = 37500
N = NUM_USERS + NUM_ITEMS
E = 800000
D = 64
DH = D // 2
NC = 2
NS = 16
LANES = 16

SUB = 128
NJ = 2
CH = SUB * NJ
NSLOT = 3
NU = 66
NCHUNK = NSLOT * NU
EPW = CH * NCHUNK
EPAD = NS * EPW

RCH = N // NS

_mesh = plsc.VectorSubcoreMesh(
    core_axis_name="c", subcore_axis_name="s", num_cores=NC, num_subcores=NS
)


def _layer_body(x_hbm, ep_hbm, out_hbm, epv, rowsv, acc, sem_g, sem_s, sem_i):
    c = lax.axis_index("c")
    s = lax.axis_index("s")
    cbase = s * NCHUNK
    xc = x_hbm.at[c]
    outc = out_hbm.at[c]

    @pl.loop(0, CH)
    def _zero_rows(r):
        for k in range(DH // LANES):
            rowsv[0, r, pl.ds(k * LANES, LANES)] = jnp.zeros(
                (LANES,), jnp.float32)

    zstart = s * RCH
    n_full, rem = RCH // CH, RCH % CH
    for z in range(n_full):
        pltpu.sync_copy(rowsv.at[0], acc.at[pl.ds(zstart + z * CH, CH)])
    if rem:
        pltpu.sync_copy(rowsv.at[0, pl.ds(0, rem)],
                        acc.at[pl.ds(zstart + n_full * CH, rem)])
    plsc.subcore_barrier()

    def fire_idx(i, p):
        pltpu.async_copy(ep_hbm.at[cbase + i], epv.at[p], sem_i[p])

    def wait_idx(p):
        pltpu.make_async_copy(ep_hbm.at[0], epv.at[p], sem_i[p]).wait()

    def fire_gather(p):
        for j in range(NJ):
            pltpu.async_copy(xc.at[epv.at[p, 0, j]],
                             rowsv.at[p, pl.ds(j * SUB, SUB)], sem_g[p])

    def wait_gather(p):
        for j in range(NJ):
            pltpu.make_async_copy(xc.at[epv.at[p, 0, j]],
                                  rowsv.at[p, pl.ds(j * SUB, SUB)],
                                  sem_g[p]).wait()

    def fire_scatter(p):
        for j in range(NJ):
            pltpu.async_copy(rowsv.at[p, pl.ds(j * SUB, SUB)],
                             acc.at[epv.at[p, 1, j]], sem_s[p], add=True)

    def wait_scatter(p):
        for j in range(NJ):
            pltpu.make_async_copy(rowsv.at[p, pl.ds(j * SUB, SUB)],
                                  acc.at[epv.at[p, 1, j]], sem_s[p]).wait()

    def compute(p):
        for j in range(NJ):
            @plsc.parallel_loop(0, SUB // LANES)
            def _group(g, j=j):
                w16 = plsc.bitcast(epv[p, 2, j, pl.ds(g * LANES, LANES)],
                                   jnp.float32)
                e0 = j * SUB + g * LANES
                for l in range(LANES):
                    wb = lax.broadcast(w16[l], (LANES,))
                    for k in range(DH // LANES):
                        sl = pl.ds(k * LANES, LANES)
                        rowsv[p, e0 + l, sl] = rowsv[p, e0 + l, sl] * wb

    def step(i, d, kind):
        p = d
        q = (d + 1) % NSLOT
        wait_gather(p)
        if not (kind == 0 and d < NSLOT - 1):
            wait_scatter(q)
        if not (kind == 2 and d == NSLOT - 1):
            wait_idx(q)
            fire_gather(q)
        compute(p)
        fire_scatter(p)
        if kind != 2:
            fire_idx(i + NSLOT, p)

    for p in range(NSLOT):
        fire_idx(p, p)
    wait_idx(0)
    fire_gather(0)
    for d in range(NSLOT):
        step(d, d, 0)

    @pl.loop(1, NU - 1)
    def _ring(u):
        for d in range(NSLOT):
            step(u * NSLOT + d, d, 1)

    for d in range(NSLOT):
        step(NCHUNK - NSLOT + d, d, 2)

    for r in range(1, NSLOT):
        wait_scatter((NCHUNK - NSLOT + r) % NSLOT)
    plsc.subcore_barrier()

    pltpu.sync_copy(acc.at[pl.ds(s * RCH, RCH)],
                    outc.at[pl.ds(s * RCH, RCH)])


def _propagate_layer(xh, epack):
    kern = pl.kernel(
        _layer_body,
        out_type=jax.ShapeDtypeStruct((NC, N, DH), jnp.float32),
        mesh=_mesh,
        scratch_types=[
            pltpu.VMEM((NSLOT, 3, NJ, SUB), jnp.int32),
            pltpu.VMEM((NSLOT, CH, DH), jnp.float32),
            pltpu.VMEM_SHARED((N, DH), jnp.float32),
            [pltpu.SemaphoreType.DMA] * NSLOT,
            [pltpu.SemaphoreType.DMA] * NSLOT,
            [pltpu.SemaphoreType.DMA] * NSLOT,
        ],
        compiler_params=pltpu.CompilerParams(
            use_tc_tiling_on_sc=False, needs_layout_passes=False),
    )
    return kern(xh, epack)


def _mean_body(a_ref, b_ref, c_ref, d_ref, o_ref):
    o_ref[...] = (a_ref[...] + b_ref[...] + c_ref[...] + d_ref[...]) * 0.25


def _mean4(x0h, x1h, x2h, x3h):
    blk = 2000
    spec = pl.BlockSpec((1, blk, DH), lambda h, i: (h, i, 0))
    return pl.pallas_call(
        _mean_body,
        grid=(NC, N // blk),
        in_specs=[spec] * 4,
        out_specs=spec,
        out_shape=jax.ShapeDtypeStruct((NC, N, DH), jnp.float32),
    )(x0h, x1h, x2h, x3h)


def kernel(user_emb, item_emb, edge_index, edge_weight):
    x0 = jnp.concatenate([user_emb, item_emb], axis=0)
    x0h = jnp.stack([x0[:, :DH], x0[:, DH:]], axis=0)
    src = edge_index[0]
    dst = edge_index[1]

    pad = EPAD - E
    ar = jnp.arange(pad, dtype=jnp.int32)
    src_p = jnp.concatenate([src, (ar * 61) % N]).reshape(-1, NJ, SUB)
    dst_p = jnp.concatenate([dst, (ar * 97) % N]).reshape(-1, NJ, SUB)
    w_p = lax.bitcast_convert_type(
        jnp.concatenate([edge_weight, jnp.zeros((pad,), jnp.float32)]),
        jnp.int32).reshape(-1, NJ, SUB)
    epack = jnp.stack([src_p, dst_p, w_p], axis=1)

    x1h = _propagate_layer(x0h, epack)
    x2h = _propagate_layer(x1h, epack)
    x3h = _propagate_layer(x2h, epack)
    fh = _mean4(x0h, x1h, x2h, x3h)
    final = jnp.concatenate([fh[0], fh[1]], axis=1)
    return final[:NUM_USERS], final[NUM_USERS:]
